# baseline (speedup 1.0000x reference)
import functools
import jax
import jax.numpy as jnp
from jax import lax
from jax.experimental import pallas as pl
from jax.experimental.pallas import tpu as pltpu
from jax.experimental.pallas import tpu_sc as plsc

_N = 10000
_E = 320000
_H = 128
_OUT = 64
_G = 128
_L = 4

_NC = 2
_NS = 16
_NW = _NC * _NS
_EPT = _E // _NW
_W = 80
_NWIN = 128
_EPTP = _NWIN * _W
_PAD = _EPTP - _EPT
_QW = 32
_NQ = _NWIN // _QW
_NA = _N + 8
_CPT = 10
_RPT = _N // _CPT


def _make_sc_segment_sum():
  mesh = plsc.VectorSubcoreMesh(core_axis_name="c", subcore_axis_name="s")

  @functools.partial(
      pl.kernel,
      out_type=jax.ShapeDtypeStruct((_NC, _N, _H), jnp.float32),
      mesh=mesh,
      scratch_types=[
          pltpu.VMEM((8, _W), jnp.int32),
          pltpu.VMEM((2, _QW, _W), jnp.int32),
          pltpu.VMEM((4, _W, _H), jnp.float32),
          pltpu.VMEM_SHARED((_NA, _H), jnp.float32),
          [pltpu.SemaphoreType.DMA] * 8,
          [pltpu.SemaphoreType.DMA] * 2,
          [pltpu.SemaphoreType.DMA] * 4,
      ],
  )
  def k(h_hbm, src_hbm, dst_hbm, out_hbm, sidx, didx, rows, acc,
        slsems, dlsems, gsems):
    core = lax.axis_index("c")
    sub = lax.axis_index("s")
    wid = sub * _NC + core
    ebase = wid * _EPTP

    @pl.when(sub < _CPT)
    def _():
      r0 = sub * _RPT
      pltpu.sync_copy(h_hbm.at[pl.ds(r0, _RPT)], acc.at[pl.ds(r0, _RPT)])
    plsc.subcore_barrier()

    def sl_issue(w, j):
      pltpu.async_copy(src_hbm.at[pl.ds(ebase + w * _W, _W)], sidx.at[j],
                       slsems[j])

    def sl_wait(j):
      pltpu.make_async_copy(src_hbm.at[pl.ds(0, _W)], sidx.at[j],
                            slsems[j]).wait()

    def dl_issue(q, sb):
      pltpu.async_copy(dst_hbm.at[wid, pl.ds(q * _QW, _QW)], didx.at[sb],
                       dlsems[sb])

    def dl_wait(sb):
      pltpu.make_async_copy(dst_hbm.at[0, pl.ds(0, _QW)], didx.at[sb],
                            dlsems[sb]).wait()

    def g_issue(w, j, b):
      pltpu.async_copy(h_hbm.at[sidx.at[j]], rows.at[b], gsems[b])

    def g_wait(b):
      pltpu.make_async_copy(h_hbm.at[sidx.at[0]], rows.at[b],
                            gsems[b]).wait()

    dl_issue(0, 0)
    dl_issue(1, 1)
    for w in range(8):
      sl_issue(w, w)
    for w in range(4):
      sl_wait(w)
      g_issue(w, w, w)

    for q in range(_NQ):
      sb = q % 2
      dl_wait(sb)
      for lw in range(_QW):
        ww = q * _QW + lw
        b = ww % 4
        j = ww % 8
        g_wait(b)
        if ww + 8 < _NWIN:
          sl_issue(ww + 8, j)
        pltpu.sync_copy(rows.at[b], acc.at[didx.at[sb, lw]], add=True)
        if ww + 4 < _NWIN:
          j4 = (ww + 4) % 8
          sl_wait(j4)
          g_issue(ww + 4, j4, b)
      if q + 2 < _NQ:
        dl_issue(q + 2, sb)

    plsc.subcore_barrier()

    @pl.when(sub < _CPT)
    def _():
      r0 = sub * _RPT
      pltpu.sync_copy(acc.at[pl.ds(r0, _RPT)],
                      out_hbm.at[core, pl.ds(r0, _RPT)])

  return k


_sc_segment_sum = _make_sc_segment_sum()


def _bn(m, g, be):
  mu = jnp.mean(m, axis=0, keepdims=True)
  var = jnp.mean((m - mu) ** 2, axis=0, keepdims=True)
  return g * (m - mu) / jnp.sqrt(var + 1e-5) + be


_HP = jax.lax.Precision.HIGHEST


def _tc_layer_body(h_ref, a_ref, w1, bb1, g1, be1, w2, bb2, g2, be2,
                   hout_ref):
  z = a_ref[0] + a_ref[1] - h_ref[...]
  m = jnp.dot(z, w1[...], precision=_HP) + bb1[...]
  m = jnp.maximum(_bn(m, g1[...], be1[...]), 0.0)
  m = jnp.dot(m, w2[...], precision=_HP) + bb2[...]
  m = jnp.maximum(_bn(m, g2[...], be2[...]), 0.0)
  hout_ref[...] = m


def _tc_layer(h, agg, p):
  return pl.pallas_call(
      _tc_layer_body,
      out_shape=jax.ShapeDtypeStruct((_N, _H), jnp.float32),
  )(h, agg,
    p['W1'], p['b1'].reshape(1, -1), p['g1'].reshape(1, -1),
    p['be1'].reshape(1, -1),
    p['W2'], p['b2'].reshape(1, -1), p['g'].reshape(1, -1),
    p['be'].reshape(1, -1))


def _tc_pool_body(b_ref, h_ref, wf, bf, y_ref, yout_ref):
  onehot = (lax.broadcasted_iota(jnp.int32, (_G, _N), 0) ==
            b_ref[...]).astype(jnp.float32)
  pooled = jnp.dot(onehot, h_ref[...], precision=_HP)
  yout_ref[...] = y_ref[...] + jnp.dot(pooled, wf[...],
                                       precision=_HP) + bf[...]


def _tc_pool(batch2d, h, fc, y):
  return pl.pallas_call(
      _tc_pool_body,
      out_shape=jax.ShapeDtypeStruct((_G, _OUT), jnp.float32),
  )(batch2d, h, fc['W'], fc['b'].reshape(1, -1), y)


def kernel(x, edge_index, batch, params):
  src = jnp.concatenate(
      [edge_index[0].reshape(_NW, _EPT),
       jnp.zeros((_NW, _PAD), jnp.int32)], axis=1).reshape(-1)
  dummy = _N + (jnp.arange(_NW, dtype=jnp.int32) % 8)[:, None]
  dst = jnp.concatenate(
      [edge_index[1].reshape(_NW, _EPT),
       jnp.broadcast_to(dummy, (_NW, _PAD))], axis=1).reshape(
           _NW, _NWIN, _W)
  batch2d = batch.reshape(1, _N)
  y = jnp.zeros((_G, _OUT), jnp.float32)
  h = x
  y = _tc_pool(batch2d, x, params['fcs'][0], y)
  for i in range(_L):
    agg = _sc_segment_sum(h, src, dst)
    h = _tc_layer(h, agg, params['conv%d' % i])
    y = _tc_pool(batch2d, h, params['fcs'][i + 1], y)
  return y


# speedup vs baseline: 3.1579x; 3.1579x over previous
import functools
import jax
import jax.numpy as jnp
from jax import lax
from jax.experimental import pallas as pl
from jax.experimental.pallas import tpu as pltpu
from jax.experimental.pallas import tpu_sc as plsc

_N = 10000
_E = 320000
_H = 128
_OUT = 64
_G = 128
_L = 4

_NC = 2
_NS = 16
_NW = _NC * _NS
_EPT = _E // _NW
_W = 80
_NWIN = 128
_EPTP = _NWIN * _W
_PAD = _EPTP - _EPT
_NREAL = _EPT // _W
_QW = 32
_NQ = _NWIN // _QW
_NA = _N
_CPT = 10
_RPT = _N // _CPT


def _make_sc_segment_sum():
  mesh = plsc.VectorSubcoreMesh(core_axis_name="c", subcore_axis_name="s")

  @functools.partial(
      pl.kernel,
      out_type=jax.ShapeDtypeStruct((_NC, _N, _H), jnp.float32),
      mesh=mesh,
      scratch_types=[
          pltpu.VMEM((8, _W), jnp.int32),
          pltpu.VMEM((2, _QW, _W), jnp.int32),
          pltpu.VMEM((4, _W, _H), jnp.float32),
          pltpu.VMEM_SHARED((_NA, _H), jnp.float32),
          [pltpu.SemaphoreType.DMA] * 8,
          [pltpu.SemaphoreType.DMA] * 2,
          [pltpu.SemaphoreType.DMA] * 4,
      ],
  )
  def k(h_hbm, src_hbm, dst_hbm, out_hbm, sidx, didx, rows, acc,
        slsems, dlsems, gsems):
    core = lax.axis_index("c")
    sub = lax.axis_index("s")
    wid = sub * _NC + core
    ebase = wid * _EPTP

    @pl.when(sub < _CPT)
    def _():
      r0 = sub * _RPT
      pltpu.sync_copy(h_hbm.at[pl.ds(r0, _RPT)], acc.at[pl.ds(r0, _RPT)])
    plsc.subcore_barrier()

    def sl_issue(w, j):
      pltpu.async_copy(src_hbm.at[pl.ds(ebase + w * _W, _W)], sidx.at[j],
                       slsems[j])

    def sl_wait(j):
      pltpu.make_async_copy(src_hbm.at[pl.ds(0, _W)], sidx.at[j],
                            slsems[j]).wait()

    def dl_issue(q, sb):
      pltpu.async_copy(dst_hbm.at[wid, pl.ds(q * _QW, _QW)], didx.at[sb],
                       dlsems[sb])

    def dl_wait(sb):
      pltpu.make_async_copy(dst_hbm.at[0, pl.ds(0, _QW)], didx.at[sb],
                            dlsems[sb]).wait()

    def g_issue(w, j, b):
      pltpu.async_copy(h_hbm.at[sidx.at[j]], rows.at[b], gsems[b])

    def g_wait(b):
      pltpu.make_async_copy(h_hbm.at[sidx.at[0]], rows.at[b],
                            gsems[b]).wait()

    dl_issue(0, 0)
    dl_issue(1, 1)
    for w in range(8):
      sl_issue(w, w)
    for w in range(4):
      sl_wait(w)
      g_issue(w, w, w)

    for q in range(_NQ):
      sb = q % 2
      dl_wait(sb)
      for lw in range(_QW):
        ww = q * _QW + lw
        if ww >= _NREAL:
          continue
        b = ww % 4
        j = ww % 8
        g_wait(b)
        if ww + 8 < _NREAL:
          sl_issue(ww + 8, j)
        pltpu.sync_copy(rows.at[b], acc.at[didx.at[sb, lw]], add=True)
        if ww + 4 < _NREAL:
          j4 = (ww + 4) % 8
          sl_wait(j4)
          g_issue(ww + 4, j4, b)
      if q + 2 < _NQ:
        dl_issue(q + 2, sb)

    plsc.subcore_barrier()

    @pl.when(sub < _CPT)
    def _():
      r0 = sub * _RPT
      pltpu.sync_copy(acc.at[pl.ds(r0, _RPT)],
                      out_hbm.at[core, pl.ds(r0, _RPT)])

  return k


_sc_segment_sum = _make_sc_segment_sum()


def _bn(m, g, be):
  mu = jnp.mean(m, axis=0, keepdims=True)
  var = jnp.mean((m - mu) ** 2, axis=0, keepdims=True)
  return g * (m - mu) / jnp.sqrt(var + 1e-5) + be


_HP = jax.lax.Precision.HIGHEST


def _tc_layer_body(h_ref, a_ref, w1, bb1, g1, be1, w2, bb2, g2, be2,
                   hout_ref):
  z = a_ref[0] + a_ref[1] - h_ref[...]
  m = jnp.dot(z, w1[...], precision=_HP) + bb1[...]
  m = jnp.maximum(_bn(m, g1[...], be1[...]), 0.0)
  m = jnp.dot(m, w2[...], precision=_HP) + bb2[...]
  m = jnp.maximum(_bn(m, g2[...], be2[...]), 0.0)
  hout_ref[...] = m


def _tc_layer(h, agg, p):
  return pl.pallas_call(
      _tc_layer_body,
      out_shape=jax.ShapeDtypeStruct((_N, _H), jnp.float32),
  )(h, agg,
    p['W1'], p['b1'].reshape(1, -1), p['g1'].reshape(1, -1),
    p['be1'].reshape(1, -1),
    p['W2'], p['b2'].reshape(1, -1), p['g'].reshape(1, -1),
    p['be'].reshape(1, -1))


def _tc_pool_body(b_ref, h_ref, wf, bf, y_ref, yout_ref):
  onehot = (lax.broadcasted_iota(jnp.int32, (_G, _N), 0) ==
            b_ref[...]).astype(jnp.float32)
  pooled = jnp.dot(onehot, h_ref[...], precision=_HP)
  yout_ref[...] = y_ref[...] + jnp.dot(pooled, wf[...],
                                       precision=_HP) + bf[...]


def _tc_pool(batch2d, h, fc, y):
  return pl.pallas_call(
      _tc_pool_body,
      out_shape=jax.ShapeDtypeStruct((_G, _OUT), jnp.float32),
  )(batch2d, h, fc['W'], fc['b'].reshape(1, -1), y)


def kernel(x, edge_index, batch, params):
  src = jnp.concatenate(
      [edge_index[0].reshape(_NW, _EPT),
       jnp.zeros((_NW, _PAD), jnp.int32)], axis=1).reshape(-1)
  dst = jnp.concatenate(
      [edge_index[1].reshape(_NW, _EPT),
       jnp.zeros((_NW, _PAD), jnp.int32)], axis=1).reshape(
           _NW, _NWIN, _W)
  batch2d = batch.reshape(1, _N)
  y = jnp.zeros((_G, _OUT), jnp.float32)
  h = x
  y = _tc_pool(batch2d, x, params['fcs'][0], y)
  for i in range(_L):
    agg = _sc_segment_sum(h, src, dst)
    h = _tc_layer(h, agg, params['conv%d' % i])
    y = _tc_pool(batch2d, h, params['fcs'][i + 1], y)
  return y
